# baseline (speedup 1.0000x reference)
import functools
import math

import jax
import jax.numpy as jnp
import numpy as np
from jax import lax
from jax.experimental import pallas as pl
from jax.experimental.pallas import tpu as pltpu
from jax.experimental.pallas import tpu_sc as plsc

NC, NS = 2, 16
NW = NC * NS
B, L, EMB = 4096, 20, 128
N = B * L
PER_W = N // NW
CHUNK = 320
NCHUNK = PER_W // CHUNK
BLK = 10240
BLK2 = BLK // 2
GRP = CHUNK // 2

_HI = np.uint32(0xFFFF0000)
_RND = np.uint32(0x8000)

_sc_mesh = plsc.VectorSubcoreMesh(core_axis_name="c", subcore_axis_name="s")


@functools.partial(
    pl.kernel,
    mesh=_sc_mesh,
    compiler_params=pltpu.CompilerParams(needs_layout_passes=False),
    out_type=jax.ShapeDtypeStruct((N // 2, EMB), jnp.uint32),
    scratch_types=[
        pltpu.VMEM((PER_W,), jnp.int32),
        pltpu.VMEM((CHUNK, EMB), jnp.float32),
        pltpu.VMEM((CHUNK, EMB), jnp.float32),
        pltpu.VMEM((CHUNK // 2, EMB), jnp.uint32),
        pltpu.VMEM((CHUNK // 2, EMB), jnp.uint32),
        pltpu.SemaphoreType.DMA,
        pltpu.SemaphoreType.DMA,
        pltpu.SemaphoreType.DMA,
        pltpu.SemaphoreType.DMA,
    ],
)
def _sc_gather(idx_hbm, table_hbm, out_hbm, idx_all, r0, r1, p0, p1,
               g0, g1, s0, s1):
    wid = lax.axis_index("s") * NC + lax.axis_index("c")
    base = wid * PER_W
    base2 = wid * (PER_W // 2)
    rows_v = (r0, r1)
    pk_v = (p0, p1)
    gsem = (g0, g1)
    ssem = (s0, s1)

    pltpu.sync_copy(idx_hbm.at[pl.ds(base, PER_W)], idx_all)

    def start_gather(i, b):
        idx_sl = idx_all.at[pl.ds(i * CHUNK, CHUNK)]
        return pltpu.async_copy(table_hbm.at[idx_sl], rows_v[b], gsem[b])

    def pack_chunk(b):
        src = rows_v[b]
        dst = pk_v[b]
        half = CHUNK // 2

        def quad_body(q, _):
            t0 = q * 8
            for dt in range(8):
                se = src.at[t0 + dt]
                so = src.at[t0 + dt + half]
                dr = dst.at[t0 + dt]
                for g in range(EMB // 16):
                    sl = pl.ds(16 * g, 16)
                    wa = plsc.bitcast(se[sl], jnp.uint32) + _RND
                    wb = plsc.bitcast(so[sl], jnp.uint32) + _RND
                    dr[sl] = (wa >> 16) | (wb & _HI)
            return ()

        lax.fori_loop(0, half // 8, quad_body, ())

    gathers = [None, None]
    scatters = [None, None]
    gathers[0] = start_gather(0, 0)
    for i in range(NCHUNK):
        b = i % 2
        nb = 1 - b
        if i + 1 < NCHUNK:
            gathers[nb] = start_gather(i + 1, nb)
        gathers[b].wait()
        if scatters[b] is not None:
            scatters[b].wait()
            scatters[b] = None
        pack_chunk(b)
        scatters[b] = pltpu.async_copy(
            pk_v[b], out_hbm.at[pl.ds(base2 + i * (CHUNK // 2), CHUNK // 2)],
            ssem[b],
        )
    for sc in scatters:
        if sc is not None:
            sc.wait()


def _gelu_mlp(x, w1, b1, w2, b2):
    h = jnp.dot(x, w1, preferred_element_type=jnp.float32) + b1
    h = 0.5 * h * (1.0 + lax.erf(h * (1.0 / math.sqrt(2.0))))
    return jnp.dot(h, w2, preferred_element_type=jnp.float32) + b2


def _mlp_body(w_ref, w1_ref, b1_ref, w2_ref, b2_ref, o_ref):
    w = w_ref[...]
    xe = lax.bitcast_convert_type(w << 16, jnp.float32)
    xo = lax.bitcast_convert_type(w & _HI, jnp.float32)
    args = (w1_ref[...], b1_ref[...], w2_ref[...], b2_ref[...])
    ye = _gelu_mlp(xe, *args)
    yo = _gelu_mlp(xo, *args)
    for k in range(BLK2 // GRP):
        o_ref[pl.ds(2 * k * GRP, GRP), :] = ye[k * GRP:(k + 1) * GRP, :]
        o_ref[pl.ds((2 * k + 1) * GRP, GRP), :] = yo[k * GRP:(k + 1) * GRP, :]


_mlp = pl.pallas_call(
    _mlp_body,
    grid=(N // BLK,),
    in_specs=[
        pl.BlockSpec((BLK2, EMB), lambda i: (i, 0)),
        pl.BlockSpec((EMB, EMB), lambda i: (0, 0)),
        pl.BlockSpec((1, EMB), lambda i: (0, 0)),
        pl.BlockSpec((EMB, EMB), lambda i: (0, 0)),
        pl.BlockSpec((1, EMB), lambda i: (0, 0)),
    ],
    out_specs=pl.BlockSpec((BLK, EMB), lambda i: (i, 0)),
    out_shape=jax.ShapeDtypeStruct((N, EMB), jnp.float32),
)


def kernel(te, E, W1, b1, W2, b2):
    idx = te.T.reshape(-1).astype(jnp.int32)
    packed = _sc_gather(idx, E)
    out = _mlp(packed, W1, b1.reshape(1, EMB), W2, b2.reshape(1, EMB))
    return out.reshape(L, B, EMB).transpose(1, 0, 2)


# speedup vs baseline: 1.0488x; 1.0488x over previous
import functools
import math

import jax
import jax.numpy as jnp
import numpy as np
from jax import lax
from jax.experimental import pallas as pl
from jax.experimental.pallas import tpu as pltpu
from jax.experimental.pallas import tpu_sc as plsc

NC, NS = 2, 16
NW = NC * NS
B, L, EMB = 4096, 20, 128
N = B * L
PER_W = N // NW
CHUNK = 320
NCHUNK = PER_W // CHUNK
BLK = 10240
BLK2 = BLK // 2
GRP = CHUNK // 2

_HI = np.uint32(0xFFFF0000)
_RND = np.uint32(0x8000)

_sc_mesh = plsc.VectorSubcoreMesh(core_axis_name="c", subcore_axis_name="s")


@functools.partial(
    pl.kernel,
    mesh=_sc_mesh,
    compiler_params=pltpu.CompilerParams(needs_layout_passes=False),
    out_type=jax.ShapeDtypeStruct((N // 2, EMB), jnp.uint32),
    scratch_types=[
        pltpu.VMEM((PER_W,), jnp.int32),
        pltpu.VMEM((CHUNK, EMB), jnp.float32),
        pltpu.VMEM((CHUNK, EMB), jnp.float32),
        pltpu.VMEM((CHUNK // 2, EMB), jnp.uint32),
        pltpu.VMEM((CHUNK // 2, EMB), jnp.uint32),
        pltpu.SemaphoreType.DMA,
        pltpu.SemaphoreType.DMA,
        pltpu.SemaphoreType.DMA,
        pltpu.SemaphoreType.DMA,
    ],
)
def _sc_gather(idx_hbm, table_hbm, out_hbm, idx_all, r0, r1, p0, p1,
               g0, g1, s0, s1):
    wid = lax.axis_index("s") * NC + lax.axis_index("c")
    base = wid * PER_W
    base2 = wid * (PER_W // 2)
    rows_v = (r0, r1)
    pk_v = (p0, p1)
    gsem = (g0, g1)
    ssem = (s0, s1)

    pltpu.sync_copy(idx_hbm.at[pl.ds(base, PER_W)], idx_all)

    def start_gather(i, b):
        idx_sl = idx_all.at[pl.ds(i * CHUNK, CHUNK)]
        return pltpu.async_copy(table_hbm.at[idx_sl], rows_v[b], gsem[b])

    def pack_chunk(b):
        src = rows_v[b]
        dst = pk_v[b]
        half = CHUNK // 2

        def quad_body(q, _):
            t0 = q * 4
            for dt in range(4):
                se = src.at[t0 + dt]
                so = src.at[t0 + dt + half]
                dr = dst.at[t0 + dt]
                for g in range(EMB // 16):
                    sl = pl.ds(16 * g, 16)
                    wa = plsc.bitcast(se[sl], jnp.uint32) + _RND
                    wb = plsc.bitcast(so[sl], jnp.uint32) + _RND
                    dr[sl] = (wa >> 16) | (wb & _HI)
            return ()

        lax.fori_loop(0, half // 4, quad_body, ())

    gathers = [None, None]
    scatters = [None, None]
    gathers[0] = start_gather(0, 0)
    for i in range(NCHUNK):
        b = i % 2
        nb = 1 - b
        if i + 1 < NCHUNK:
            gathers[nb] = start_gather(i + 1, nb)
        gathers[b].wait()
        if scatters[b] is not None:
            scatters[b].wait()
            scatters[b] = None
        pack_chunk(b)
        scatters[b] = pltpu.async_copy(
            pk_v[b], out_hbm.at[pl.ds(base2 + i * (CHUNK // 2), CHUNK // 2)],
            ssem[b],
        )
    for sc in scatters:
        if sc is not None:
            sc.wait()


def _gelu_mlp(x, w1, b1, w2, b2):
    h = jnp.dot(x, w1, preferred_element_type=jnp.float32) + b1
    h = 0.5 * h * (1.0 + lax.erf(h * (1.0 / math.sqrt(2.0))))
    return jnp.dot(h, w2, preferred_element_type=jnp.float32) + b2


def _mlp_body(w_ref, w1_ref, b1_ref, w2_ref, b2_ref, o_ref):
    w = w_ref[...]
    xe = lax.bitcast_convert_type(w << 16, jnp.float32)
    xo = lax.bitcast_convert_type(w & _HI, jnp.float32)
    args = (w1_ref[...], b1_ref[...], w2_ref[...], b2_ref[...])
    ye = _gelu_mlp(xe, *args)
    yo = _gelu_mlp(xo, *args)
    for k in range(BLK2 // GRP):
        o_ref[pl.ds(2 * k * GRP, GRP), :] = ye[k * GRP:(k + 1) * GRP, :]
        o_ref[pl.ds((2 * k + 1) * GRP, GRP), :] = yo[k * GRP:(k + 1) * GRP, :]


_mlp = pl.pallas_call(
    _mlp_body,
    grid=(N // BLK,),
    in_specs=[
        pl.BlockSpec((BLK2, EMB), lambda i: (i, 0)),
        pl.BlockSpec((EMB, EMB), lambda i: (0, 0)),
        pl.BlockSpec((1, EMB), lambda i: (0, 0)),
        pl.BlockSpec((EMB, EMB), lambda i: (0, 0)),
        pl.BlockSpec((1, EMB), lambda i: (0, 0)),
    ],
    out_specs=pl.BlockSpec((BLK, EMB), lambda i: (i, 0)),
    out_shape=jax.ShapeDtypeStruct((N, EMB), jnp.float32),
)


def kernel(te, E, W1, b1, W2, b2):
    idx = te.T.reshape(-1).astype(jnp.int32)
    packed = _sc_gather(idx, E)
    out = _mlp(packed, W1, b1.reshape(1, EMB), W2, b2.reshape(1, EMB))
    return out.reshape(L, B, EMB).transpose(1, 0, 2)


# speedup vs baseline: 1.0653x; 1.0158x over previous
import functools
import math

import jax
import jax.numpy as jnp
import numpy as np
from jax import lax
from jax.experimental import pallas as pl
from jax.experimental.pallas import tpu as pltpu
from jax.experimental.pallas import tpu_sc as plsc

NC, NS = 2, 16
NW = NC * NS
B, L, EMB = 4096, 20, 128
N = B * L
PER_W = N // NW
CHUNK = 320
NCHUNK = PER_W // CHUNK
BLK = 20480
BLK2 = BLK // 2
GRP = CHUNK // 2

_HI = np.uint32(0xFFFF0000)
_RND = np.uint32(0x8000)

_sc_mesh = plsc.VectorSubcoreMesh(core_axis_name="c", subcore_axis_name="s")


@functools.partial(
    pl.kernel,
    mesh=_sc_mesh,
    compiler_params=pltpu.CompilerParams(needs_layout_passes=False),
    out_type=jax.ShapeDtypeStruct((N // 2, EMB), jnp.uint32),
    scratch_types=[
        pltpu.VMEM((PER_W,), jnp.int32),
        pltpu.VMEM((CHUNK, EMB), jnp.float32),
        pltpu.VMEM((CHUNK, EMB), jnp.float32),
        pltpu.VMEM((CHUNK // 2, EMB), jnp.uint32),
        pltpu.VMEM((CHUNK // 2, EMB), jnp.uint32),
        pltpu.SemaphoreType.DMA,
        pltpu.SemaphoreType.DMA,
        pltpu.SemaphoreType.DMA,
        pltpu.SemaphoreType.DMA,
    ],
)
def _sc_gather(idx_hbm, table_hbm, out_hbm, idx_all, r0, r1, p0, p1,
               g0, g1, s0, s1):
    wid = lax.axis_index("s") * NC + lax.axis_index("c")
    base = wid * PER_W
    base2 = wid * (PER_W // 2)
    rows_v = (r0, r1)
    pk_v = (p0, p1)
    gsem = (g0, g1)
    ssem = (s0, s1)

    pltpu.sync_copy(idx_hbm.at[pl.ds(base, PER_W)], idx_all)

    def start_gather(i, b):
        idx_sl = idx_all.at[pl.ds(i * CHUNK, CHUNK)]
        return pltpu.async_copy(table_hbm.at[idx_sl], rows_v[b], gsem[b])

    def pack_chunk(b):
        src = rows_v[b]
        dst = pk_v[b]
        half = CHUNK // 2

        def quad_body(q, _):
            t0 = q * 4
            for dt in range(4):
                se = src.at[t0 + dt]
                so = src.at[t0 + dt + half]
                dr = dst.at[t0 + dt]
                for g in range(EMB // 16):
                    sl = pl.ds(16 * g, 16)
                    wa = plsc.bitcast(se[sl], jnp.uint32) + _RND
                    wb = plsc.bitcast(so[sl], jnp.uint32) + _RND
                    dr[sl] = (wa >> 16) | (wb & _HI)
            return ()

        lax.fori_loop(0, half // 4, quad_body, ())

    gathers = [None, None]
    scatters = [None, None]
    gathers[0] = start_gather(0, 0)
    for i in range(NCHUNK):
        b = i % 2
        nb = 1 - b
        if i + 1 < NCHUNK:
            gathers[nb] = start_gather(i + 1, nb)
        gathers[b].wait()
        if scatters[b] is not None:
            scatters[b].wait()
            scatters[b] = None
        pack_chunk(b)
        scatters[b] = pltpu.async_copy(
            pk_v[b], out_hbm.at[pl.ds(base2 + i * (CHUNK // 2), CHUNK // 2)],
            ssem[b],
        )
    for sc in scatters:
        if sc is not None:
            sc.wait()


def _gelu_mlp(x, w1, b1, w2, b2):
    h = jnp.dot(x, w1, preferred_element_type=jnp.float32) + b1
    h = 0.5 * h * (1.0 + lax.erf(h * (1.0 / math.sqrt(2.0))))
    return jnp.dot(h, w2, preferred_element_type=jnp.float32) + b2


def _mlp_body(w_ref, w1_ref, b1_ref, w2_ref, b2_ref, o_ref):
    w = w_ref[...]
    xe = lax.bitcast_convert_type(w << 16, jnp.float32)
    xo = lax.bitcast_convert_type(w & _HI, jnp.float32)
    args = (w1_ref[...], b1_ref[...], w2_ref[...], b2_ref[...])
    ye = _gelu_mlp(xe, *args)
    yo = _gelu_mlp(xo, *args)
    for k in range(BLK2 // GRP):
        o_ref[pl.ds(2 * k * GRP, GRP), :] = ye[k * GRP:(k + 1) * GRP, :]
        o_ref[pl.ds((2 * k + 1) * GRP, GRP), :] = yo[k * GRP:(k + 1) * GRP, :]


_mlp = pl.pallas_call(
    _mlp_body,
    grid=(N // BLK,),
    in_specs=[
        pl.BlockSpec((BLK2, EMB), lambda i: (i, 0)),
        pl.BlockSpec((EMB, EMB), lambda i: (0, 0)),
        pl.BlockSpec((1, EMB), lambda i: (0, 0)),
        pl.BlockSpec((EMB, EMB), lambda i: (0, 0)),
        pl.BlockSpec((1, EMB), lambda i: (0, 0)),
    ],
    out_specs=pl.BlockSpec((BLK, EMB), lambda i: (i, 0)),
    out_shape=jax.ShapeDtypeStruct((N, EMB), jnp.float32),
)


def kernel(te, E, W1, b1, W2, b2):
    idx = te.T.reshape(-1).astype(jnp.int32)
    packed = _sc_gather(idx, E)
    out = _mlp(packed, W1, b1.reshape(1, EMB), W2, b2.reshape(1, EMB))
    return out.reshape(L, B, EMB).transpose(1, 0, 2)
